# baseline (speedup 1.0000x reference)
import functools

import jax
import jax.numpy as jnp
from jax import lax
from jax.experimental import pallas as pl
from jax.experimental.pallas import tpu as pltpu
from jax.experimental.pallas import tpu_sc as plsc

_NC = 2
_NS = 16
_NW = _NC * _NS
_BPG = 2
_NSLOT = 2


def _make_scatter_kernel(B, T, D):
    TPW = T // _NW
    G = B // _BPG
    ROWS = _BPG * TPW

    mesh = plsc.VectorSubcoreMesh(core_axis_name="c", subcore_axis_name="s")

    @functools.partial(
        pl.kernel,
        out_type=jax.ShapeDtypeStruct((B * T, D), jnp.float32),
        mesh=mesh,
        scratch_types=[
            pltpu.VMEM((TPW,), jnp.int32),
            pltpu.VMEM((G, ROWS), jnp.int32),
            pltpu.VMEM((_NSLOT, ROWS, D), jnp.float32),
            pltpu.SemaphoreType.DMA,
            pltpu.SemaphoreType.DMA,
        ],
    )
    def scatter_kernel(img_hbm, idx_hbm, out_hbm,
                       rawidx_v, flatidx_v, buf_v, sem_in, sem_out):
        c = lax.axis_index("c")
        s = lax.axis_index("s")
        wid = s * _NC + c
        base = wid * TPW

        def in_copy(g, slot, half):
            b = g * _BPG + half
            return pltpu.make_async_copy(
                img_hbm.at[pl.ds(b * T + base, TPW)],
                buf_v.at[slot, pl.ds(half * TPW, TPW)], sem_in)

        def out_copy(g, slot):
            return pltpu.make_async_copy(
                buf_v.at[slot], out_hbm.at[flatidx_v.at[g]], sem_out)

        for j in range(_NSLOT):
            for h in range(_BPG):
                in_copy(j, j, h).start()

        pltpu.sync_copy(idx_hbm.at[pl.ds(base, TPW)], rawidx_v)

        def fill(g, carry):
            for q in range(0, ROWS, 16):
                flatidx_v[g, pl.ds(q, 16)] = (
                    rawidx_v[pl.ds(q % TPW, 16)] + (g * _BPG + q // TPW) * T)
            return carry
        lax.fori_loop(0, G, fill, 0)

        def step(g, carry):
            for j in range(_NSLOT):
                gg = g * _NSLOT + j
                for h in range(_BPG):
                    in_copy(gg, j, h).wait()
                out_copy(gg, j).start()
                out_copy(gg, j).wait()
                for h in range(_BPG):
                    in_copy(gg + _NSLOT, j, h).start()
            return carry
        lax.fori_loop(0, G // _NSLOT - 1, step, 0)

        glast = G - _NSLOT
        for j in range(_NSLOT):
            for h in range(_BPG):
                in_copy(glast + j, j, h).wait()
            out_copy(glast + j, j).start()
        for j in range(_NSLOT):
            out_copy(glast + j, j).wait()

    return scatter_kernel


def kernel(img, index_flat_inv):
    B, T, D = img.shape
    img_flat = img.reshape(B * T, D)
    idx = index_flat_inv.astype(jnp.int32)
    out_flat = _make_scatter_kernel(B, T, D)(img_flat, idx)
    return out_flat.reshape(B, T, D)
